# Initial kernel scaffold; baseline (speedup 1.0000x reference)
import jax
import jax.numpy as jnp
from jax.experimental import pallas as pl


def kernel(user_ids, item_ids, explicit_feedback, dense_weight_slice):
    raise NotImplementedError("write your pallas kernel here")


# speedup vs baseline: 1.9228x; 1.9228x over previous
import jax
import jax.numpy as jnp
from jax.experimental import pallas as pl


def _mm_kernel(a_ref, w_ref, o_ref):
    w = jnp.maximum(w_ref[...], 0.0).astype(jnp.bfloat16)
    a = a_ref[...].astype(jnp.bfloat16)
    o_ref[...] = jnp.dot(a, w, preferred_element_type=jnp.float32)


def kernel(user_ids, item_ids, explicit_feedback, dense_weight_slice):
    M, K = explicit_feedback.shape
    N = dense_weight_slice.shape[1]
    bm = 512
    return pl.pallas_call(
        _mm_kernel,
        grid=(M // bm,),
        in_specs=[
            pl.BlockSpec((bm, K), lambda i: (i, 0)),
            pl.BlockSpec((K, N), lambda i: (0, 0)),
        ],
        out_specs=pl.BlockSpec((bm, N), lambda i: (i, 0)),
        out_shape=jax.ShapeDtypeStruct((M, N), jnp.float32),
    )(explicit_feedback, dense_weight_slice)
